# baseline (speedup 1.0000x reference)
import functools

import jax
import jax.numpy as jnp
from jax import lax
from jax.experimental import pallas as pl
from jax.experimental.pallas import tpu as pltpu
from jax.experimental.pallas import tpu_sc as plsc

_SCORE_THRESH = 0.7
_TOP_K_KP = 512
_TOP_K_M = 128


def _masked_scores_kernel(rep_ref, rel_ref, out_ref):
    x = rep_ref[0]
    rel = rel_ref[0]
    H, W = x.shape
    neginf = jnp.float32(-jnp.inf)
    row_pad = jnp.full((1, W), neginf, dtype=x.dtype)
    up = jnp.concatenate([x[1:], row_pad], axis=0)
    dn = jnp.concatenate([row_pad, x[:-1]], axis=0)
    vmax = jnp.maximum(jnp.maximum(x, up), dn)
    col_pad = jnp.full((H, 1), neginf, dtype=x.dtype)
    lf = jnp.concatenate([vmax[:, 1:], col_pad], axis=1)
    rt = jnp.concatenate([col_pad, vmax[:, :-1]], axis=1)
    mx = jnp.maximum(jnp.maximum(vmax, lf), rt)
    ms = jnp.sqrt(x * rel)
    keep = (x == mx) & (ms >= _SCORE_THRESH)
    out_ref[0] = jnp.where(keep, ms, neginf)


def _masked_scores(rep, rel):
    B = rep.shape[0]
    H, W = rep.shape[2], rep.shape[3]
    r = rep.reshape(B, H, W)
    l = rel.reshape(B, H, W)
    out = pl.pallas_call(
        _masked_scores_kernel,
        grid=(B,),
        in_specs=[
            pl.BlockSpec((1, H, W), lambda b: (b, 0, 0)),
            pl.BlockSpec((1, H, W), lambda b: (b, 0, 0)),
        ],
        out_specs=pl.BlockSpec((1, H, W), lambda b: (b, 0, 0)),
        out_shape=jax.ShapeDtypeStruct((B, H, W), jnp.float32),
    )(r, l)
    return out.reshape(B, H * W)


def _match_kernel(d1_ref, s1_ref, d2_ref, s2_ref, out_ref):
    a = d1_ref[0] * s1_ref[0]
    b = d2_ref[0] * s2_ref[0]
    out_ref[0] = jax.lax.dot_general(
        a, b, dimension_numbers=(((0,), (0,)), ((), ())),
        preferred_element_type=jnp.float32)


def _match_scores(d1, s1, d2, s2):
    B, C, K = d1.shape
    L = d2.shape[2]
    return pl.pallas_call(
        _match_kernel,
        grid=(B,),
        in_specs=[
            pl.BlockSpec((1, C, K), lambda b: (b, 0, 0)),
            pl.BlockSpec((1, 1, K), lambda b: (b, 0, 0)),
            pl.BlockSpec((1, C, L), lambda b: (b, 0, 0)),
            pl.BlockSpec((1, 1, L), lambda b: (b, 0, 0)),
        ],
        out_specs=pl.BlockSpec((1, K, L), lambda b: (b, 0, 0)),
        out_shape=jax.ShapeDtypeStruct((B, K, L), jnp.float32),
    )(d1, s1.reshape(B, 1, K), d2, s2.reshape(B, 1, L))


_CHUNK = 18432


def _gather_cols(desc, idx):
    B, C, n = desc.shape
    K = idx.shape[1]
    if n <= _CHUNK:
        return jnp.take_along_axis(
            desc, jnp.broadcast_to(idx[:, None, :], (B, C, K)), axis=2)
    nch = n // _CHUNK
    d4 = desc.reshape(B, C, nch, _CHUNK)
    ch = idx // _CHUNK
    local = idx - ch * _CHUNK
    idx4 = jnp.broadcast_to(local[:, None, None, :], (B, C, nch, K))
    g4 = jnp.take_along_axis(d4, idx4, axis=3)
    mask = (ch[:, None, :] == jnp.arange(nch, dtype=jnp.int32)[None, :, None])
    m4 = mask.astype(jnp.float32)[:, None, :, :]
    return (g4 * m4).sum(axis=2)


def _extract_scores(rep0, rel0, rep1, rel1):
    H0, W0 = rep0.shape[2], rep0.shape[3]
    H1, W1 = rep1.shape[2], rep1.shape[3]
    n0 = H0 * W0
    n1 = H1 * W1

    s0 = _masked_scores(rep0, rel0)
    s1 = _masked_scores(rep1, rel1)
    scores_all = jnp.concatenate([s0, s1], axis=1)
    top_scores, idx = jax.lax.top_k(scores_all, _TOP_K_KP)

    in0 = idx < n0
    j1 = idx - n0
    r = jnp.where(in0, idx // W0, 2 * (j1 // W1))
    c = jnp.where(in0, idx % W0, 2 * (j1 % W1))
    loc = jnp.stack([r, c], axis=1).astype(jnp.float32)

    i0 = jnp.clip(idx, 0, n0 - 1)
    i1 = jnp.clip(j1, 0, n1 - 1)
    return top_scores, loc, in0, i0, i1


def kernel(img1_rep_s0, img1_rel_s0, img1_desc_s0, img1_rep_s1, img1_rel_s1,
           img1_desc_s1, img2_rep_s0, img2_rel_s0, img2_desc_s0, img2_rep_s1,
           img2_rel_s1, img2_desc_s1):
    B = img1_rep_s0.shape[0]
    C = img1_desc_s0.shape[1]
    n0 = img1_rep_s0.shape[2] * img1_rep_s0.shape[3]
    n1 = img1_rep_s1.shape[2] * img1_rep_s1.shape[3]

    sc1, loc1, in0_1, i0_1, i1_1 = _extract_scores(
        img1_rep_s0, img1_rel_s0, img1_rep_s1, img1_rel_s1)
    sc2, loc2, in0_2, i0_2, i1_2 = _extract_scores(
        img2_rep_s0, img2_rel_s0, img2_rep_s1, img2_rel_s1)

    d1 = jnp.where(in0_1[:, None, :],
                   _gather_cols(img1_desc_s0.reshape(B, C, n0), i0_1),
                   _gather_cols(img1_desc_s1.reshape(B, C, n1), i1_1))
    d2 = jnp.where(in0_2[:, None, :],
                   _gather_cols(img2_desc_s0.reshape(B, C, n0), i0_2),
                   _gather_cols(img2_desc_s1.reshape(B, C, n1), i1_2))
    scores = _match_scores(d1, sc1, d2, sc2)
    B, K, L = scores.shape
    _, top_idx = jax.lax.top_k(scores.reshape(B, K * L), _TOP_K_M)
    i1 = top_idx // L
    i2 = top_idx % L
    m1 = jnp.take_along_axis(
        loc1, jnp.broadcast_to(i1[:, None, :], (B, 2, _TOP_K_M)), axis=2)
    m2 = jnp.take_along_axis(
        loc2, jnp.broadcast_to(i2[:, None, :], (B, 2, _TOP_K_M)), axis=2)
    return jnp.concatenate([m1, m2], axis=1)


# speedup vs baseline: 1.1502x; 1.1502x over previous
import functools

import jax
import jax.numpy as jnp
from jax import lax
from jax.experimental import pallas as pl
from jax.experimental.pallas import tpu as pltpu
from jax.experimental.pallas import tpu_sc as plsc

_SCORE_THRESH = 0.7
_TOP_K_KP = 512
_TOP_K_M = 128


def _masked_scores_kernel(rep_ref, rel_ref, out_ref):
    x = rep_ref[0]
    rel = rel_ref[0]
    H, W = x.shape
    neginf = jnp.float32(-jnp.inf)
    row_pad = jnp.full((1, W), neginf, dtype=x.dtype)
    up = jnp.concatenate([x[1:], row_pad], axis=0)
    dn = jnp.concatenate([row_pad, x[:-1]], axis=0)
    vmax = jnp.maximum(jnp.maximum(x, up), dn)
    col_pad = jnp.full((H, 1), neginf, dtype=x.dtype)
    lf = jnp.concatenate([vmax[:, 1:], col_pad], axis=1)
    rt = jnp.concatenate([col_pad, vmax[:, :-1]], axis=1)
    mx = jnp.maximum(jnp.maximum(vmax, lf), rt)
    ms = jnp.sqrt(x * rel)
    keep = (x == mx) & (ms >= _SCORE_THRESH)
    out_ref[0] = jnp.where(keep, ms, neginf)


def _masked_scores(rep, rel):
    B = rep.shape[0]
    H, W = rep.shape[2], rep.shape[3]
    r = rep.reshape(B, H, W)
    l = rel.reshape(B, H, W)
    out = pl.pallas_call(
        _masked_scores_kernel,
        grid=(B,),
        in_specs=[
            pl.BlockSpec((1, H, W), lambda b: (b, 0, 0)),
            pl.BlockSpec((1, H, W), lambda b: (b, 0, 0)),
        ],
        out_specs=pl.BlockSpec((1, H, W), lambda b: (b, 0, 0)),
        out_shape=jax.ShapeDtypeStruct((B, H, W), jnp.float32),
    )(r, l)
    return out.reshape(B, H * W)


def _match_kernel(d1_ref, s1_ref, d2_ref, s2_ref, out_ref):
    a = d1_ref[0] * s1_ref[0]
    b = d2_ref[0] * s2_ref[0]
    out_ref[0] = jax.lax.dot_general(
        a, b, dimension_numbers=(((0,), (0,)), ((), ())),
        preferred_element_type=jnp.float32)


def _match_scores(d1, s1, d2, s2):
    B, C, K = d1.shape
    L = d2.shape[2]
    return pl.pallas_call(
        _match_kernel,
        grid=(B,),
        in_specs=[
            pl.BlockSpec((1, C, K), lambda b: (b, 0, 0)),
            pl.BlockSpec((1, 1, K), lambda b: (b, 0, 0)),
            pl.BlockSpec((1, C, L), lambda b: (b, 0, 0)),
            pl.BlockSpec((1, 1, L), lambda b: (b, 0, 0)),
        ],
        out_specs=pl.BlockSpec((1, K, L), lambda b: (b, 0, 0)),
        out_shape=jax.ShapeDtypeStruct((B, K, L), jnp.float32),
    )(d1, s1.reshape(B, 1, K), d2, s2.reshape(B, 1, L))


_CHUNK = 36864


def _topk512(scores_all):
    B, N = scores_all.shape
    nch = 6
    m = N // nch
    s = scores_all.reshape(B, nch, m)
    v, li = jax.lax.top_k(s, _TOP_K_KP)
    gi = li + jnp.arange(nch, dtype=li.dtype)[None, :, None] * m
    v2, p = jax.lax.top_k(v.reshape(B, nch * _TOP_K_KP), _TOP_K_KP)
    idx = jnp.take_along_axis(gi.reshape(B, nch * _TOP_K_KP), p, axis=1)
    return v2, idx


def _gather_cols(desc, idx):
    B, C, n = desc.shape
    K = idx.shape[1]
    if n <= _CHUNK:
        return jnp.take_along_axis(
            desc, jnp.broadcast_to(idx[:, None, :], (B, C, K)), axis=2)
    nch = n // _CHUNK
    d4 = desc.reshape(B, C, nch, _CHUNK)
    ch = idx // _CHUNK
    local = idx - ch * _CHUNK
    idx4 = jnp.broadcast_to(local[:, None, None, :], (B, C, nch, K))
    g4 = jnp.take_along_axis(d4, idx4, axis=3)
    mask = (ch[:, None, :] == jnp.arange(nch, dtype=jnp.int32)[None, :, None])
    m4 = mask.astype(jnp.float32)[:, None, :, :]
    return (g4 * m4).sum(axis=2)


def _extract_scores(rep0, rel0, rep1, rel1):
    H0, W0 = rep0.shape[2], rep0.shape[3]
    H1, W1 = rep1.shape[2], rep1.shape[3]
    n0 = H0 * W0
    n1 = H1 * W1

    s0 = _masked_scores(rep0, rel0)
    s1 = _masked_scores(rep1, rel1)
    scores_all = jnp.concatenate([s0, s1], axis=1)
    top_scores, idx = _topk512(scores_all)

    in0 = idx < n0
    j1 = idx - n0
    r = jnp.where(in0, idx // W0, 2 * (j1 // W1))
    c = jnp.where(in0, idx % W0, 2 * (j1 % W1))
    loc = jnp.stack([r, c], axis=1).astype(jnp.float32)

    i0 = jnp.clip(idx, 0, n0 - 1)
    i1 = jnp.clip(j1, 0, n1 - 1)
    return top_scores, loc, in0, i0, i1


def kernel(img1_rep_s0, img1_rel_s0, img1_desc_s0, img1_rep_s1, img1_rel_s1,
           img1_desc_s1, img2_rep_s0, img2_rel_s0, img2_desc_s0, img2_rep_s1,
           img2_rel_s1, img2_desc_s1):
    B = img1_rep_s0.shape[0]
    C = img1_desc_s0.shape[1]
    n0 = img1_rep_s0.shape[2] * img1_rep_s0.shape[3]
    n1 = img1_rep_s1.shape[2] * img1_rep_s1.shape[3]

    sc1, loc1, in0_1, i0_1, i1_1 = _extract_scores(
        img1_rep_s0, img1_rel_s0, img1_rep_s1, img1_rel_s1)
    sc2, loc2, in0_2, i0_2, i1_2 = _extract_scores(
        img2_rep_s0, img2_rel_s0, img2_rep_s1, img2_rel_s1)

    d1 = jnp.where(in0_1[:, None, :],
                   _gather_cols(img1_desc_s0.reshape(B, C, n0), i0_1),
                   _gather_cols(img1_desc_s1.reshape(B, C, n1), i1_1))
    d2 = jnp.where(in0_2[:, None, :],
                   _gather_cols(img2_desc_s0.reshape(B, C, n0), i0_2),
                   _gather_cols(img2_desc_s1.reshape(B, C, n1), i1_2))
    scores = _match_scores(d1, sc1, d2, sc2)
    B, K, L = scores.shape
    _, top_idx = jax.lax.top_k(scores.reshape(B, K * L), _TOP_K_M)
    i1 = top_idx // L
    i2 = top_idx % L
    m1 = jnp.take_along_axis(
        loc1, jnp.broadcast_to(i1[:, None, :], (B, 2, _TOP_K_M)), axis=2)
    m2 = jnp.take_along_axis(
        loc2, jnp.broadcast_to(i2[:, None, :], (B, 2, _TOP_K_M)), axis=2)
    return jnp.concatenate([m1, m2], axis=1)
